# baseline (device time: 51899 ns/iter reference)
import jax
import jax.numpy as jnp
from jax import lax
from jax.experimental import pallas as pl
from jax.experimental.pallas import tpu as pltpu

N_DEV = 16
B, SQ, SKV, HQ, DH = 2, 256, 256, 64, 64
H_LOC = HQ // N_DEV
D_MODEL = 512
CHUNK = SQ // N_DEV
NEG = -1e9


def _block_mask():
    qb = lax.broadcasted_iota(jnp.int32, (SQ, SKV), 0) // 64
    kb = lax.broadcasted_iota(jnp.int32, (SQ, SKV), 1) // 64
    return (qb == kb) | (kb == 0) | ((qb + kb) % 3 == 0)


def kernel(x, Wq, K_ext, V_ext, Wo):
    def body(x_ref, wq_ref, k_hbm, v_hbm, wo_ref, out_ref,
             part_ref, red_ref, rs_buf, k_ref, v_ref,
             rs_send, rs_recv, ag_send, ag_recv, kv_sem):
        my = lax.axis_index("i")

        k_cp = pltpu.make_async_copy(
            k_hbm.at[:, :, pl.ds(my * H_LOC, H_LOC), :], k_ref, kv_sem.at[0])
        v_cp = pltpu.make_async_copy(
            v_hbm.at[:, :, pl.ds(my * H_LOC, H_LOC), :], v_ref, kv_sem.at[1])
        k_cp.start()
        v_cp.start()

        barrier = pltpu.get_barrier_semaphore()
        for j in range(N_DEV):
            pl.semaphore_signal(barrier, inc=1, device_id=(j,),
                                device_id_type=pl.DeviceIdType.MESH)
        k_cp.wait()
        v_cp.wait()

        mask = _block_mask()
        wq = (wq_ref[...] * 0.125).astype(jnp.bfloat16)
        wo = wo_ref[...].astype(jnp.bfloat16)
        for b in range(B):
            q = jnp.dot(x_ref[b].astype(jnp.bfloat16), wq,
                        preferred_element_type=jnp.float32)
            q = q.astype(jnp.bfloat16)
            ctx = []
            for h in range(H_LOC):
                qh = q[:, h * DH:(h + 1) * DH]
                s = lax.dot_general(
                    qh, k_ref[b, :, h, :].astype(jnp.bfloat16),
                    (((1,), (1,)), ((), ())),
                    preferred_element_type=jnp.float32)
                w = jnp.exp(jnp.where(mask, s, NEG))
                w = w / jnp.sum(w, axis=1, keepdims=True)
                ctx.append(jnp.dot(w.astype(jnp.bfloat16),
                                   v_ref[b, :, h, :].astype(jnp.bfloat16),
                                   preferred_element_type=jnp.float32))
            ctx = jnp.concatenate(ctx, axis=1).astype(jnp.bfloat16)
            part_ref[b] = jnp.dot(ctx, wo,
                                  preferred_element_type=jnp.float32
                                  ).astype(jnp.bfloat16)

            if b == 0:
                pl.semaphore_wait(barrier, N_DEV)
            for j in range(N_DEV):
                @pl.when(my != j)
                def _(b=b, j=j):
                    pltpu.make_async_remote_copy(
                        src_ref=part_ref.at[b, pl.ds(j * CHUNK, CHUNK), :],
                        dst_ref=rs_buf.at[my, b],
                        send_sem=rs_send.at[j, b],
                        recv_sem=rs_recv.at[my, b],
                        device_id=(j,),
                        device_id_type=pl.DeviceIdType.MESH,
                    ).start()

        for b in range(B):
            for j in range(N_DEV):
                @pl.when(my != j)
                def _(b=b, j=j):
                    pltpu.make_async_remote_copy(
                        src_ref=rs_buf.at[j, b], dst_ref=rs_buf.at[j, b],
                        send_sem=rs_send.at[j, b], recv_sem=rs_recv.at[j, b],
                        device_id=(j,), device_id_type=pl.DeviceIdType.MESH,
                    ).wait_recv()

            acc = part_ref[b, pl.ds(my * CHUNK, CHUNK), :].astype(jnp.float32)
            for j in range(N_DEV):
                val = rs_buf[j, b].astype(jnp.float32)
                acc = acc + jnp.where(my == j, jnp.zeros_like(val), val)
            red = acc.astype(jnp.bfloat16)

            out_ref[b, pl.ds(my * CHUNK, CHUNK), :] = red
            red_ref[b] = red

            for j in range(N_DEV):
                @pl.when(my != j)
                def _(b=b, j=j):
                    pltpu.make_async_remote_copy(
                        src_ref=red_ref.at[b],
                        dst_ref=out_ref.at[b, pl.ds(my * CHUNK, CHUNK), :],
                        send_sem=ag_send.at[j, b],
                        recv_sem=ag_recv.at[my, b],
                        device_id=(j,),
                        device_id_type=pl.DeviceIdType.MESH,
                    ).start()

        for b in range(B):
            for j in range(N_DEV):
                @pl.when(my != j)
                def _(b=b, j=j):
                    pltpu.make_async_remote_copy(
                        src_ref=red_ref.at[b],
                        dst_ref=out_ref.at[b, pl.ds(j * CHUNK, CHUNK), :],
                        send_sem=ag_send.at[j, b],
                        recv_sem=ag_recv.at[j, b],
                        device_id=(j,), device_id_type=pl.DeviceIdType.MESH,
                    ).wait_recv()

        for j in range(N_DEV):
            @pl.when(my != j)
            def _(j=j):
                for b in range(B):
                    pltpu.make_async_remote_copy(
                        src_ref=part_ref.at[b, pl.ds(j * CHUNK, CHUNK), :],
                        dst_ref=rs_buf.at[j, b],
                        send_sem=rs_send.at[j, b], recv_sem=rs_recv.at[j, b],
                        device_id=(j,), device_id_type=pl.DeviceIdType.MESH,
                    ).wait_send()
                    pltpu.make_async_remote_copy(
                        src_ref=red_ref.at[b],
                        dst_ref=out_ref.at[b, pl.ds(j * CHUNK, CHUNK), :],
                        send_sem=ag_send.at[j, b], recv_sem=ag_recv.at[j, b],
                        device_id=(j,), device_id_type=pl.DeviceIdType.MESH,
                    ).wait_send()

    return pl.pallas_call(
        body,
        out_shape=jax.ShapeDtypeStruct((B, SQ, D_MODEL), jnp.bfloat16),
        in_specs=[
            pl.BlockSpec(memory_space=pltpu.VMEM),
            pl.BlockSpec(memory_space=pltpu.VMEM),
            pl.BlockSpec(memory_space=pl.ANY),
            pl.BlockSpec(memory_space=pl.ANY),
            pl.BlockSpec(memory_space=pltpu.VMEM),
        ],
        out_specs=pl.BlockSpec(memory_space=pltpu.VMEM),
        scratch_shapes=[
            pltpu.VMEM((B, SQ, D_MODEL), jnp.bfloat16),
            pltpu.VMEM((B, CHUNK, D_MODEL), jnp.bfloat16),
            pltpu.VMEM((N_DEV, B, CHUNK, D_MODEL), jnp.bfloat16),
            pltpu.VMEM((B, SQ, H_LOC, DH), jnp.float32),
            pltpu.VMEM((B, SQ, H_LOC, DH), jnp.float32),
            pltpu.SemaphoreType.DMA((N_DEV, B)),
            pltpu.SemaphoreType.DMA((N_DEV, B)),
            pltpu.SemaphoreType.DMA((N_DEV, B)),
            pltpu.SemaphoreType.DMA((N_DEV, B)),
            pltpu.SemaphoreType.DMA((2,)),
        ],
        compiler_params=pltpu.CompilerParams(collective_id=0),
    )(x, Wq, K_ext, V_ext, Wo)


# device time: 29912 ns/iter; 1.7351x vs baseline; 1.7351x over previous
import jax
import jax.numpy as jnp
from jax import lax
from jax.experimental import pallas as pl
from jax.experimental.pallas import tpu as pltpu

N_DEV = 16
B, SQ, SKV, HQ, DH = 2, 256, 256, 64, 64
H_LOC = HQ // N_DEV
D_MODEL = 512
ROWS = 32
PER_B = SQ // ROWS
NEG = -1e9


def _chunk(j):
    return j // PER_B, (j % PER_B) * ROWS


def _block_mask():
    qb = lax.broadcasted_iota(jnp.int32, (SQ, SKV), 0) // 64
    kb = lax.broadcasted_iota(jnp.int32, (SQ, SKV), 1) // 64
    return (qb == kb) | (kb == 0) | ((qb + kb) % 3 == 0)


def kernel(x, Wq, K_ext, V_ext, Wo):
    idx = lax.axis_index("i")
    k_loc = lax.dynamic_slice_in_dim(K_ext, idx * H_LOC, H_LOC, axis=2)
    v_loc = lax.dynamic_slice_in_dim(V_ext, idx * H_LOC, H_LOC, axis=2)
    k_loc = k_loc.astype(jnp.bfloat16)
    v_loc = v_loc.astype(jnp.bfloat16)

    def body(x_ref, wq_ref, k_ref, v_ref, wo_ref, out_ref,
             part_ref, red_ref, rs_buf,
             rs_send, rs_recv, ag_send, ag_recv):
        my = lax.axis_index("i")
        my_b = my // PER_B
        my_r = (my % PER_B) * ROWS

        barrier = pltpu.get_barrier_semaphore()
        for j in range(N_DEV):
            pl.semaphore_signal(barrier, inc=1, device_id=(j,),
                                device_id_type=pl.DeviceIdType.MESH)

        mask = _block_mask()
        wq = (wq_ref[...] * 0.125).astype(jnp.bfloat16)
        wo = wo_ref[...].astype(jnp.bfloat16)
        for b in range(B):
            q = jnp.dot(x_ref[b].astype(jnp.bfloat16), wq,
                        preferred_element_type=jnp.float32)
            q = q.astype(jnp.bfloat16)
            ctx = []
            for h in range(H_LOC):
                qh = q[:, h * DH:(h + 1) * DH]
                s = lax.dot_general(
                    qh, k_ref[b, :, h, :],
                    (((1,), (1,)), ((), ())),
                    preferred_element_type=jnp.float32)
                w = jnp.exp(jnp.where(mask, s, NEG))
                w = w / jnp.sum(w, axis=1, keepdims=True)
                ctx.append(jnp.dot(w.astype(jnp.bfloat16), v_ref[b, :, h, :],
                                   preferred_element_type=jnp.float32))
            ctx = jnp.concatenate(ctx, axis=1).astype(jnp.bfloat16)
            part_ref[b] = jnp.dot(ctx, wo,
                                  preferred_element_type=jnp.float32
                                  ).astype(jnp.bfloat16)

            if b == 0:
                pl.semaphore_wait(barrier, N_DEV)
            for j in range(N_DEV):
                bj, rj = _chunk(j)
                if bj != b:
                    continue
                @pl.when(my != j)
                def _(bj=bj, rj=rj, j=j):
                    pltpu.make_async_remote_copy(
                        src_ref=part_ref.at[bj, pl.ds(rj, ROWS), :],
                        dst_ref=rs_buf.at[my],
                        send_sem=rs_send.at[j],
                        recv_sem=rs_recv.at[my],
                        device_id=(j,),
                        device_id_type=pl.DeviceIdType.MESH,
                    ).start()

        for j in range(N_DEV):
            @pl.when(my != j)
            def _(j=j):
                pltpu.make_async_remote_copy(
                    src_ref=rs_buf.at[j], dst_ref=rs_buf.at[j],
                    send_sem=rs_send.at[j], recv_sem=rs_recv.at[j],
                    device_id=(j,), device_id_type=pl.DeviceIdType.MESH,
                ).wait_recv()

        acc = part_ref[my_b, pl.ds(my_r, ROWS), :].astype(jnp.float32)
        for j in range(N_DEV):
            val = rs_buf[j].astype(jnp.float32)
            acc = acc + jnp.where(my == j, jnp.zeros_like(val), val)
        red = acc.astype(jnp.bfloat16)

        out_ref[my_b, pl.ds(my_r, ROWS), :] = red
        red_ref[...] = red

        for j in range(N_DEV):
            @pl.when(my != j)
            def _(j=j):
                pltpu.make_async_remote_copy(
                    src_ref=red_ref,
                    dst_ref=out_ref.at[my_b, pl.ds(my_r, ROWS), :],
                    send_sem=ag_send.at[j],
                    recv_sem=ag_recv.at[my],
                    device_id=(j,),
                    device_id_type=pl.DeviceIdType.MESH,
                ).start()

        for j in range(N_DEV):
            bj, rj = _chunk(j)
            @pl.when(my != j)
            def _(bj=bj, rj=rj, j=j):
                pltpu.make_async_remote_copy(
                    src_ref=red_ref,
                    dst_ref=out_ref.at[bj, pl.ds(rj, ROWS), :],
                    send_sem=ag_send.at[j], recv_sem=ag_recv.at[j],
                    device_id=(j,), device_id_type=pl.DeviceIdType.MESH,
                ).wait_recv()

        for j in range(N_DEV):
            bj, rj = _chunk(j)
            @pl.when(my != j)
            def _(bj=bj, rj=rj, j=j):
                pltpu.make_async_remote_copy(
                    src_ref=part_ref.at[bj, pl.ds(rj, ROWS), :],
                    dst_ref=rs_buf.at[j],
                    send_sem=rs_send.at[j], recv_sem=rs_recv.at[j],
                    device_id=(j,), device_id_type=pl.DeviceIdType.MESH,
                ).wait_send()
                pltpu.make_async_remote_copy(
                    src_ref=red_ref,
                    dst_ref=out_ref.at[bj, pl.ds(rj, ROWS), :],
                    send_sem=ag_send.at[j], recv_sem=ag_recv.at[j],
                    device_id=(j,), device_id_type=pl.DeviceIdType.MESH,
                ).wait_send()

    return pl.pallas_call(
        body,
        out_shape=jax.ShapeDtypeStruct((B, SQ, D_MODEL), jnp.bfloat16),
        in_specs=[pl.BlockSpec(memory_space=pltpu.VMEM)] * 5,
        out_specs=pl.BlockSpec(memory_space=pltpu.VMEM),
        scratch_shapes=[
            pltpu.VMEM((B, SQ, D_MODEL), jnp.bfloat16),
            pltpu.VMEM((ROWS, D_MODEL), jnp.bfloat16),
            pltpu.VMEM((N_DEV, ROWS, D_MODEL), jnp.bfloat16),
            pltpu.SemaphoreType.DMA((N_DEV,)),
            pltpu.SemaphoreType.DMA((N_DEV,)),
            pltpu.SemaphoreType.DMA((N_DEV,)),
            pltpu.SemaphoreType.DMA((N_DEV,)),
        ],
        compiler_params=pltpu.CompilerParams(collective_id=0),
    )(x, Wq, k_loc, v_loc, Wo)


# device time: 25906 ns/iter; 2.0034x vs baseline; 1.1546x over previous
import jax
import jax.numpy as jnp
from jax import lax
from jax.experimental import pallas as pl
from jax.experimental.pallas import tpu as pltpu

N_DEV = 16
B, SQ, SKV, HQ, DH = 2, 256, 256, 64, 64
H_LOC = HQ // N_DEV
D_MODEL = 512
CHUNK = SQ // N_DEV
NEG = -1e9


def _block_mask():
    qb = lax.broadcasted_iota(jnp.int32, (SQ, SKV), 0) // 64
    kb = lax.broadcasted_iota(jnp.int32, (SQ, SKV), 1) // 64
    return (qb == kb) | (kb == 0) | ((qb + kb) % 3 == 0)


def kernel(x, Wq, K_ext, V_ext, Wo):
    idx = lax.axis_index("i")
    k_loc = lax.dynamic_slice_in_dim(K_ext, idx * H_LOC, H_LOC, axis=2)
    v_loc = lax.dynamic_slice_in_dim(V_ext, idx * H_LOC, H_LOC, axis=2)
    k_loc = k_loc.astype(jnp.bfloat16)
    v_loc = v_loc.astype(jnp.bfloat16)

    def body(x_ref, wq_ref, k_ref, v_ref, wo_ref, out_ref,
             part_ref, red_ref, rs_buf,
             rs_send, rs_recv, ag_send, ag_recv):
        my = lax.axis_index("i")

        barrier = pltpu.get_barrier_semaphore()
        for j in range(N_DEV):
            @pl.when(my != j)
            def _(j=j):
                pl.semaphore_signal(barrier, inc=1, device_id=(j,),
                                    device_id_type=pl.DeviceIdType.MESH)

        mask = _block_mask()
        wq = (wq_ref[...] * 0.125).astype(jnp.bfloat16)
        wo = wo_ref[...].astype(jnp.bfloat16)
        for b in range(B):
            q = jnp.dot(x_ref[b].astype(jnp.bfloat16), wq,
                        preferred_element_type=jnp.float32)
            q = q.astype(jnp.bfloat16)
            ctx = []
            for h in range(H_LOC):
                qh = q[:, h * DH:(h + 1) * DH]
                s = lax.dot_general(
                    qh, k_ref[b, :, h, :],
                    (((1,), (1,)), ((), ())),
                    preferred_element_type=jnp.float32)
                w = jnp.exp(jnp.where(mask, s, NEG))
                w = w / jnp.sum(w, axis=1, keepdims=True)
                ctx.append(jnp.dot(w.astype(jnp.bfloat16), v_ref[b, :, h, :],
                                   preferred_element_type=jnp.float32))
            ctx = jnp.concatenate(ctx, axis=1).astype(jnp.bfloat16)
            part_ref[b] = jnp.dot(ctx, wo,
                                  preferred_element_type=jnp.float32
                                  ).astype(jnp.bfloat16)

            if b == 0:
                pl.semaphore_wait(barrier, N_DEV - 1)
            for j in range(N_DEV):
                @pl.when(my != j)
                def _(b=b, j=j):
                    pltpu.make_async_remote_copy(
                        src_ref=part_ref.at[b, pl.ds(j * CHUNK, CHUNK), :],
                        dst_ref=rs_buf.at[my, b],
                        send_sem=rs_send.at[j, b],
                        recv_sem=rs_recv.at[my, b],
                        device_id=(j,),
                        device_id_type=pl.DeviceIdType.MESH,
                    ).start()

        for b in range(B):
            for j in range(N_DEV):
                @pl.when(my != j)
                def _(b=b, j=j):
                    pltpu.make_async_remote_copy(
                        src_ref=rs_buf.at[j, b], dst_ref=rs_buf.at[j, b],
                        send_sem=rs_send.at[j, b], recv_sem=rs_recv.at[j, b],
                        device_id=(j,), device_id_type=pl.DeviceIdType.MESH,
                    ).wait_recv()

            acc = part_ref[b, pl.ds(my * CHUNK, CHUNK), :].astype(jnp.float32)
            for j in range(N_DEV):
                val = rs_buf[j, b].astype(jnp.float32)
                acc = acc + jnp.where(my == j, jnp.zeros_like(val), val)
            red = acc.astype(jnp.bfloat16)

            out_ref[b, pl.ds(my * CHUNK, CHUNK), :] = red
            red_ref[b] = red

            for j in range(N_DEV):
                @pl.when(my != j)
                def _(b=b, j=j):
                    pltpu.make_async_remote_copy(
                        src_ref=red_ref.at[b],
                        dst_ref=out_ref.at[b, pl.ds(my * CHUNK, CHUNK), :],
                        send_sem=ag_send.at[j, b],
                        recv_sem=ag_recv.at[my, b],
                        device_id=(j,),
                        device_id_type=pl.DeviceIdType.MESH,
                    ).start()

        for b in range(B):
            for j in range(N_DEV):
                @pl.when(my != j)
                def _(b=b, j=j):
                    pltpu.make_async_remote_copy(
                        src_ref=red_ref.at[b],
                        dst_ref=out_ref.at[b, pl.ds(j * CHUNK, CHUNK), :],
                        send_sem=ag_send.at[j, b],
                        recv_sem=ag_recv.at[j, b],
                        device_id=(j,), device_id_type=pl.DeviceIdType.MESH,
                    ).wait_recv()

        for j in range(N_DEV):
            @pl.when(my != j)
            def _(j=j):
                for b in range(B):
                    pltpu.make_async_remote_copy(
                        src_ref=part_ref.at[b, pl.ds(j * CHUNK, CHUNK), :],
                        dst_ref=rs_buf.at[j, b],
                        send_sem=rs_send.at[j, b], recv_sem=rs_recv.at[j, b],
                        device_id=(j,), device_id_type=pl.DeviceIdType.MESH,
                    ).wait_send()
                    pltpu.make_async_remote_copy(
                        src_ref=red_ref.at[b],
                        dst_ref=out_ref.at[b, pl.ds(j * CHUNK, CHUNK), :],
                        send_sem=ag_send.at[j, b], recv_sem=ag_recv.at[j, b],
                        device_id=(j,), device_id_type=pl.DeviceIdType.MESH,
                    ).wait_send()

    return pl.pallas_call(
        body,
        out_shape=jax.ShapeDtypeStruct((B, SQ, D_MODEL), jnp.bfloat16),
        in_specs=[pl.BlockSpec(memory_space=pltpu.VMEM)] * 5,
        out_specs=pl.BlockSpec(memory_space=pltpu.VMEM),
        scratch_shapes=[
            pltpu.VMEM((B, SQ, D_MODEL), jnp.bfloat16),
            pltpu.VMEM((B, CHUNK, D_MODEL), jnp.bfloat16),
            pltpu.VMEM((N_DEV, B, CHUNK, D_MODEL), jnp.bfloat16),
            pltpu.SemaphoreType.DMA((N_DEV, B)),
            pltpu.SemaphoreType.DMA((N_DEV, B)),
            pltpu.SemaphoreType.DMA((N_DEV, B)),
            pltpu.SemaphoreType.DMA((N_DEV, B)),
        ],
        compiler_params=pltpu.CompilerParams(collective_id=0),
    )(x, Wq, k_loc, v_loc, Wo)


# device time: 25718 ns/iter; 2.0180x vs baseline; 1.0073x over previous
import jax
import jax.numpy as jnp
from jax import lax
from jax.experimental import pallas as pl
from jax.experimental.pallas import tpu as pltpu

N_DEV = 16
B, SQ, SKV, HQ, DH = 2, 256, 256, 64, 64
H_LOC = HQ // N_DEV
D_HEADS = H_LOC * DH
D_MODEL = 512
CHUNK = SQ // N_DEV
NEG = -1e9


def _block_mask():
    qb = lax.broadcasted_iota(jnp.int32, (SQ, SKV), 0) // 64
    kb = lax.broadcasted_iota(jnp.int32, (SQ, SKV), 1) // 64
    return (qb == kb) | (kb == 0) | ((qb + kb) % 3 == 0)


def kernel(x, Wq, K_ext, V_ext, Wo):
    idx = lax.axis_index("i")
    k_loc = lax.dynamic_slice_in_dim(K_ext, idx * H_LOC, H_LOC, axis=2)
    v_loc = lax.dynamic_slice_in_dim(V_ext, idx * H_LOC, H_LOC, axis=2)
    k_loc = k_loc.astype(jnp.bfloat16).reshape(B, SKV, D_HEADS)
    v_loc = v_loc.astype(jnp.bfloat16).reshape(B, SKV, D_HEADS)

    def body(x_hbm, wq_hbm, k_hbm, v_hbm, wo_hbm, out_ref,
             x_s, wq_s, k_s, v_s, wo_s, part_ref, red_ref, rs_buf,
             rs_send, rs_recv, ag_send, ag_recv, in_sem, st_sem):
        my = lax.axis_index("i")

        loads = [
            pltpu.make_async_copy(x_hbm, x_s, in_sem.at[0]),
            pltpu.make_async_copy(wq_hbm, wq_s, in_sem.at[1]),
            pltpu.make_async_copy(k_hbm, k_s, in_sem.at[2]),
            pltpu.make_async_copy(v_hbm, v_s, in_sem.at[3]),
            pltpu.make_async_copy(wo_hbm, wo_s, in_sem.at[4]),
        ]
        for cp in loads:
            cp.start()

        barrier = pltpu.get_barrier_semaphore()
        for j in range(N_DEV):
            @pl.when(my != j)
            def _(j=j):
                pl.semaphore_signal(barrier, inc=1, device_id=(j,),
                                    device_id_type=pl.DeviceIdType.MESH)
        for cp in loads:
            cp.wait()

        mask = _block_mask()
        wq = (wq_s[...] * 0.125).astype(jnp.bfloat16)
        wo = wo_s[...].astype(jnp.bfloat16)
        for b in range(B):
            q = jnp.dot(x_s[b].astype(jnp.bfloat16), wq,
                        preferred_element_type=jnp.float32)
            q = q.astype(jnp.bfloat16)
            ctx = []
            for h in range(H_LOC):
                qh = q[:, h * DH:(h + 1) * DH]
                s = lax.dot_general(
                    qh, k_s[b][:, h * DH:(h + 1) * DH],
                    (((1,), (1,)), ((), ())),
                    preferred_element_type=jnp.float32)
                w = jnp.exp(jnp.where(mask, s, NEG))
                w = w / jnp.sum(w, axis=1, keepdims=True)
                ctx.append(jnp.dot(w.astype(jnp.bfloat16),
                                   v_s[b][:, h * DH:(h + 1) * DH],
                                   preferred_element_type=jnp.float32))
            ctx = jnp.concatenate(ctx, axis=1).astype(jnp.bfloat16)
            part_ref[b] = jnp.dot(ctx, wo,
                                  preferred_element_type=jnp.float32
                                  ).astype(jnp.bfloat16)

            if b == 0:
                pl.semaphore_wait(barrier, N_DEV - 1)
            for j in range(N_DEV):
                @pl.when(my != j)
                def _(b=b, j=j):
                    pltpu.make_async_remote_copy(
                        src_ref=part_ref.at[b, pl.ds(j * CHUNK, CHUNK), :],
                        dst_ref=rs_buf.at[my, b],
                        send_sem=rs_send.at[j, b],
                        recv_sem=rs_recv.at[my, b],
                        device_id=(j,),
                        device_id_type=pl.DeviceIdType.MESH,
                    ).start()

        for b in range(B):
            for j in range(N_DEV):
                @pl.when(my != j)
                def _(b=b, j=j):
                    pltpu.make_async_remote_copy(
                        src_ref=rs_buf.at[j, b], dst_ref=rs_buf.at[j, b],
                        send_sem=rs_send.at[j, b], recv_sem=rs_recv.at[j, b],
                        device_id=(j,), device_id_type=pl.DeviceIdType.MESH,
                    ).wait_recv()

            acc = part_ref[b, pl.ds(my * CHUNK, CHUNK), :].astype(jnp.float32)
            for j in range(N_DEV):
                val = rs_buf[j, b].astype(jnp.float32)
                acc = acc + jnp.where(my == j, jnp.zeros_like(val), val)
            red_ref[b] = acc.astype(jnp.bfloat16)

            pltpu.make_async_copy(
                red_ref.at[b],
                out_ref.at[b, pl.ds(my * CHUNK, CHUNK), :],
                st_sem.at[b],
            ).start()

            for j in range(N_DEV):
                @pl.when(my != j)
                def _(b=b, j=j):
                    pltpu.make_async_remote_copy(
                        src_ref=red_ref.at[b],
                        dst_ref=out_ref.at[b, pl.ds(my * CHUNK, CHUNK), :],
                        send_sem=ag_send.at[j, b],
                        recv_sem=ag_recv.at[my, b],
                        device_id=(j,),
                        device_id_type=pl.DeviceIdType.MESH,
                    ).start()

        for b in range(B):
            for j in range(N_DEV):
                @pl.when(my != j)
                def _(b=b, j=j):
                    pltpu.make_async_remote_copy(
                        src_ref=red_ref.at[b],
                        dst_ref=out_ref.at[b, pl.ds(j * CHUNK, CHUNK), :],
                        send_sem=ag_send.at[j, b],
                        recv_sem=ag_recv.at[j, b],
                        device_id=(j,), device_id_type=pl.DeviceIdType.MESH,
                    ).wait_recv()

        for b in range(B):
            pltpu.make_async_copy(
                red_ref.at[b],
                out_ref.at[b, pl.ds(my * CHUNK, CHUNK), :],
                st_sem.at[b],
            ).wait()
        for j in range(N_DEV):
            @pl.when(my != j)
            def _(j=j):
                for b in range(B):
                    pltpu.make_async_remote_copy(
                        src_ref=part_ref.at[b, pl.ds(j * CHUNK, CHUNK), :],
                        dst_ref=rs_buf.at[j, b],
                        send_sem=rs_send.at[j, b], recv_sem=rs_recv.at[j, b],
                        device_id=(j,), device_id_type=pl.DeviceIdType.MESH,
                    ).wait_send()
                    pltpu.make_async_remote_copy(
                        src_ref=red_ref.at[b],
                        dst_ref=out_ref.at[b, pl.ds(j * CHUNK, CHUNK), :],
                        send_sem=ag_send.at[j, b], recv_sem=ag_recv.at[j, b],
                        device_id=(j,), device_id_type=pl.DeviceIdType.MESH,
                    ).wait_send()

    return pl.pallas_call(
        body,
        out_shape=jax.ShapeDtypeStruct((B, SQ, D_MODEL), jnp.bfloat16),
        in_specs=[pl.BlockSpec(memory_space=pl.ANY)] * 5,
        out_specs=pl.BlockSpec(memory_space=pl.ANY),
        scratch_shapes=[
            pltpu.VMEM((B, SQ, D_MODEL), jnp.float32),
            pltpu.VMEM((D_MODEL, D_HEADS), jnp.float32),
            pltpu.VMEM((B, SKV, D_HEADS), jnp.bfloat16),
            pltpu.VMEM((B, SKV, D_HEADS), jnp.bfloat16),
            pltpu.VMEM((D_HEADS, D_MODEL), jnp.float32),
            pltpu.VMEM((B, SQ, D_MODEL), jnp.bfloat16),
            pltpu.VMEM((B, CHUNK, D_MODEL), jnp.bfloat16),
            pltpu.VMEM((N_DEV, B, CHUNK, D_MODEL), jnp.bfloat16),
            pltpu.SemaphoreType.DMA((N_DEV, B)),
            pltpu.SemaphoreType.DMA((N_DEV, B)),
            pltpu.SemaphoreType.DMA((N_DEV, B)),
            pltpu.SemaphoreType.DMA((N_DEV, B)),
            pltpu.SemaphoreType.DMA((5,)),
            pltpu.SemaphoreType.DMA((B,)),
        ],
        compiler_params=pltpu.CompilerParams(collective_id=0),
    )(x, Wq, k_loc, v_loc, Wo)
